# Initial kernel scaffold; baseline (speedup 1.0000x reference)
import functools

import jax
import jax.numpy as jnp
from jax import lax
from jax.experimental import pallas as pl
from jax.experimental.pallas import tpu as pltpu
from jax.experimental.pallas import tpu_sc as plsc

N = 10000
E = 320000
D = 128
D_OUT = 64
NP = 10240
NC = 2
NS = 16
NW = NC * NS
EPW = E // NW
K = 80
C = EPW // K
ROWS_PER_TILE = NP // NS


def _sc_kernel(src3, dst3, w3, h):
    mesh = plsc.VectorSubcoreMesh(
        core_axis_name="c", subcore_axis_name="s", num_cores=NC, num_subcores=NS
    )

    @functools.partial(
        pl.kernel,
        out_type=(
            jax.ShapeDtypeStruct((NC, NP, D), jnp.float32),
            jax.ShapeDtypeStruct((NP,), jnp.float32),
        ),
        mesh=mesh,
        scratch_types=[
            pltpu.VMEM((C, K), jnp.int32),
            pltpu.VMEM((C, K), jnp.int32),
            pltpu.VMEM((C, K), jnp.float32),
            pltpu.VMEM((K, D), jnp.float32),
            pltpu.VMEM((K, D), jnp.float32),
            pltpu.VMEM((NP,), jnp.float32),
            pltpu.VMEM((ROWS_PER_TILE,), jnp.float32),
            pltpu.VMEM((64, D), jnp.float32),
            pltpu.VMEM_SHARED((NP,), jnp.float32),
            pltpu.VMEM_SHARED((NP,), jnp.float32),
            pltpu.VMEM_SHARED((NP, D), jnp.float32),
            pltpu.SemaphoreType.DMA,
            pltpu.SemaphoreType.DMA,
            pltpu.SemaphoreType.DMA,
            pltpu.SemaphoreType.DMA,
            pltpu.SemaphoreType.DMA,
        ],
    )
    def k(src3_h, dst3_h, w3_h, h_hbm, s_out, dis_out,
          src_v, dst_v, w_v, rows_a, rows_b, dis_v, nbuf_v, zbuf,
          deg_sh, dis_sh, s_sh, semg_a, semg_b, sems_a, sems_b, sem_deg):
        c = lax.axis_index("c")
        s = lax.axis_index("s")
        wid = s * NC + c
        base = s * ROWS_PER_TILE

        z16 = jnp.zeros((16,), jnp.float32)

        def zero_nbuf(i, carry):
            nbuf_v[pl.ds(i * 16, 16)] = z16
            return carry
        lax.fori_loop(0, ROWS_PER_TILE // 16, zero_nbuf, 0)
        pltpu.sync_copy(nbuf_v, deg_sh.at[pl.ds(base, ROWS_PER_TILE)])

        def zero_zbuf(i, carry):
            for t in range(D // 16):
                zbuf[i, pl.ds(t * 16, 16)] = z16
            return carry
        lax.fori_loop(0, 64, zero_zbuf, 0)
        for kk in range(ROWS_PER_TILE // 64):
            pltpu.sync_copy(zbuf, s_sh.at[pl.ds(base + kk * 64, 64)])
        plsc.subcore_barrier()

        for t in range(2):
            slab = s * 2 + t
            pltpu.sync_copy(dst3_h.at[slab], dst_v)
            pltpu.sync_copy(w3_h.at[slab], w_v)

            def deg_group(g, carry):
                for u in range(5):
                    j = g * 5 + u
                    pltpu.async_copy(
                        w_v.at[j], deg_sh.at[dst_v.at[j]], sem_deg, add=True)
                for u in range(5):
                    j = g * 5 + u
                    pltpu.make_async_copy(
                        w_v.at[j], deg_sh.at[dst_v.at[j]], sem_deg).wait()
                return carry
            lax.fori_loop(0, C // 5, deg_group, 0)
        plsc.subcore_barrier()

        pltpu.sync_copy(deg_sh.at[pl.ds(base, ROWS_PER_TILE)], nbuf_v)

        def newton(i, carry):
            d = nbuf_v[pl.ds(i * 16, 16)] + 1.0
            bits = plsc.bitcast(d, jnp.int32)
            bits = jnp.int32(0x5F3759DF) - lax.shift_right_logical(bits, 1)
            y = plsc.bitcast(bits, jnp.float32)
            for _ in range(3):
                y = y * (1.5 - 0.5 * d * y * y)
            nbuf_v[pl.ds(i * 16, 16)] = y
            return carry
        lax.fori_loop(0, ROWS_PER_TILE // 16, newton, 0)
        pltpu.sync_copy(nbuf_v, dis_sh.at[pl.ds(base, ROWS_PER_TILE)])

        @pl.when(c == 0)
        def _():
            pltpu.sync_copy(nbuf_v, dis_out.at[pl.ds(base, ROWS_PER_TILE)])

        plsc.subcore_barrier()
        pltpu.sync_copy(dis_sh, dis_v)

        pltpu.sync_copy(src3_h.at[wid], src_v)
        pltpu.sync_copy(dst3_h.at[wid], dst_v)
        pltpu.sync_copy(w3_h.at[wid], w_v)

        def fold(i, carry):
            j = i // (K // 16)
            kk = i % (K // 16)
            s16 = src_v[j, pl.ds(kk * 16, 16)]
            wv = w_v[j, pl.ds(kk * 16, 16)]
            dv = plsc.load_gather(dis_v, [s16])
            w_v[j, pl.ds(kk * 16, 16)] = wv * dv
            return carry
        lax.fori_loop(0, C * (K // 16), fold, 0)

        def g_start(j, buf, sem):
            pltpu.async_copy(h_hbm.at[src_v.at[j]], buf, sem)

        def g_wait(j, buf, sem):
            pltpu.make_async_copy(h_hbm.at[src_v.at[j]], buf, sem).wait()

        def scale(j, buf):
            def row(i, carry):
                ii = jnp.full((16,), i, jnp.int32)
                jj = jnp.full((16,), j, jnp.int32)
                cb = plsc.load_gather(w_v, [jj, ii])
                for t in range(D // 16):
                    buf[i, pl.ds(t * 16, 16)] = buf[i, pl.ds(t * 16, 16)] * cb
                return carry
            lax.fori_loop(0, K, row, 0)

        def s_start(j, buf, sem):
            pltpu.async_copy(buf, s_sh.at[dst_v.at[j]], sem, add=True)

        def s_wait(j, buf, sem):
            pltpu.make_async_copy(buf, s_sh.at[dst_v.at[j]], sem).wait()

        g_start(0, rows_a, semg_a)

        def main(t, carry):
            j0 = 2 * t
            j1 = 2 * t + 1
            g_start(j1, rows_b, semg_b)
            g_wait(j0, rows_a, semg_a)
            scale(j0, rows_a)
            s_start(j0, rows_a, sems_a)
            g_wait(j1, rows_b, semg_b)
            scale(j1, rows_b)
            s_start(j1, rows_b, sems_b)
            s_wait(j0, rows_a, sems_a)
            g_start(j0 + 2, rows_a, semg_a)
            s_wait(j1, rows_b, sems_b)
            return carry
        lax.fori_loop(0, (C - 1) // 2, main, 0)

        jl = C - 1
        g_wait(jl, rows_a, semg_a)
        scale(jl, rows_a)
        s_start(jl, rows_a, sems_a)
        s_wait(jl, rows_a, sems_a)
        plsc.subcore_barrier()

        pltpu.sync_copy(s_sh.at[pl.ds(base, ROWS_PER_TILE)],
                        s_out.at[c, pl.ds(base, ROWS_PER_TILE)])

    return k(src3, dst3, w3, h)


def _mm_h(x_pad, W_conv):
    def body(x_ref, w_ref, o_ref):
        o_ref[...] = jnp.dot(x_ref[...], w_ref[...],
                             preferred_element_type=jnp.float32)

    return pl.pallas_call(
        body,
        grid=(NP // 256,),
        in_specs=[
            pl.BlockSpec((256, D), lambda i: (i, 0)),
            pl.BlockSpec((D, D), lambda i: (0, 0)),
        ],
        out_specs=pl.BlockSpec((256, D), lambda i: (i, 0)),
        out_shape=jax.ShapeDtypeStruct((NP, D), jnp.float32),
    )(x_pad, W_conv)


def _epilogue(s0, s1, h, dis, b_conv, W_out, b_out):
    def body(s0_ref, s1_ref, h_ref, dis_ref, bc_ref, wo_ref, bo_ref, o_ref):
        dv = dis_ref[...]
        agg = dv * (s0_ref[...] + s1_ref[...]) + (dv * dv) * h_ref[...] \
            + bc_ref[...]
        emb = jnp.where(agg >= 0, agg, 0.01 * agg)
        o_ref[...] = jnp.dot(emb, wo_ref[...],
                             preferred_element_type=jnp.float32) + bo_ref[...]

    return pl.pallas_call(
        body,
        grid=(NP // 256,),
        in_specs=[
            pl.BlockSpec((256, D), lambda i: (i, 0)),
            pl.BlockSpec((256, D), lambda i: (i, 0)),
            pl.BlockSpec((256, D), lambda i: (i, 0)),
            pl.BlockSpec((256, 1), lambda i: (i, 0)),
            pl.BlockSpec((1, D), lambda i: (0, 0)),
            pl.BlockSpec((D, D_OUT), lambda i: (0, 0)),
            pl.BlockSpec((1, D_OUT), lambda i: (0, 0)),
        ],
        out_specs=pl.BlockSpec((256, D_OUT), lambda i: (i, 0)),
        out_shape=jax.ShapeDtypeStruct((NP, D_OUT), jnp.float32),
    )(s0, s1, h, dis, b_conv, W_out, b_out)


def kernel(x, edge_index, edge_weight, W_conv, b_conv, W_out, b_out):
    src = edge_index[0].astype(jnp.int32)
    dst = edge_index[1].astype(jnp.int32)
    w = edge_weight.astype(jnp.float32)

    src3 = src.reshape(NW, C, K)
    dst3 = dst.reshape(NW, C, K)
    w3 = w.reshape(NW, C, K)
    x_pad = jnp.pad(x, ((0, NP - N), (0, 0)))

    h = _mm_h(x_pad, W_conv)
    s_part, dis = _sc_kernel(src3, dst3, w3, h)
    out = _epilogue(
        s_part[0], s_part[1], h, dis.reshape(NP, 1),
        b_conv.reshape(1, D), W_out, b_out.reshape(1, D_OUT))
    return out[:N]


# speedup vs baseline: 25.3723x; 25.3723x over previous
import functools

import jax
import jax.numpy as jnp
from jax import lax
from jax.experimental import pallas as pl
from jax.experimental.pallas import tpu as pltpu
from jax.experimental.pallas import tpu_sc as plsc

N = 10000
E = 320000
D = 128
D_OUT = 64
NC = 2
NS = 16
NW = NC * NS
EPW = E // NW
K = 80
C = EPW // K
GC = 25
NG = C // GC
NPD = 10240
DEG_ROWS = NPD // NS
S_ROWS = NPD // NS

_MESH = dict(core_axis_name="c", subcore_axis_name="s",
             num_cores=NC, num_subcores=NS)
_PARAMS = None


def _sc_params():
    return pltpu.CompilerParams(needs_layout_passes=False)


def _deg_kernel(dst3, w3):
    @functools.partial(
        pl.kernel,
        out_type=jax.ShapeDtypeStruct((NC, NPD), jnp.float32),
        mesh=plsc.VectorSubcoreMesh(**_MESH),
        compiler_params=_sc_params(),
        scratch_types=[
            pltpu.VMEM((NG, GC, K), jnp.int32),
            pltpu.VMEM((NG, GC, K), jnp.float32),
            pltpu.VMEM((DEG_ROWS,), jnp.float32),
            pltpu.VMEM_SHARED((NPD,), jnp.float32),
            pltpu.SemaphoreType.DMA,
        ],
    )
    def k(dst3_h, w3_h, deg_out, dst_v, w_v, nbuf, deg_sh, sem):
        c = lax.axis_index("c")
        s = lax.axis_index("s")
        wid = s * NC + c
        base = s * DEG_ROWS

        z16 = jnp.zeros((16,), jnp.float32)

        def zero_nbuf(i, carry):
            nbuf[pl.ds(i * 16, 16)] = z16
            return carry
        lax.fori_loop(0, DEG_ROWS // 16, zero_nbuf, 0)
        pltpu.sync_copy(nbuf, deg_sh.at[pl.ds(base, DEG_ROWS)])

        pltpu.sync_copy(dst3_h.at[wid], dst_v)
        pltpu.sync_copy(w3_h.at[wid], w_v)
        plsc.subcore_barrier()

        def deg_group(i, carry):
            g = i // 5
            t = i % 5
            for u in range(5):
                j = t * 5 + u
                pltpu.async_copy(
                    w_v.at[g, j], deg_sh.at[dst_v.at[g, j]], sem, add=True)
            for u in range(5):
                j = t * 5 + u
                pltpu.make_async_copy(
                    w_v.at[g, j], deg_sh.at[dst_v.at[g, j]], sem).wait()
            return carry
        lax.fori_loop(0, C // 5, deg_group, 0)
        plsc.subcore_barrier()

        pltpu.sync_copy(deg_sh.at[pl.ds(base, DEG_ROWS)],
                        deg_out.at[c, pl.ds(base, DEG_ROWS)])

    return k(dst3, w3)


def _agg_kernel(src3, dst3, w3, hp):
    @functools.partial(
        pl.kernel,
        out_type=jax.ShapeDtypeStruct((NC, NPD, D), jnp.float32),
        mesh=plsc.VectorSubcoreMesh(**_MESH),
        compiler_params=_sc_params(),
        scratch_types=[
            pltpu.VMEM((GC, K), jnp.int32),
            pltpu.VMEM((GC, K), jnp.int32),
            pltpu.VMEM((GC, K), jnp.float32),
            pltpu.VMEM((K, D), jnp.float32),
            pltpu.VMEM((K, D), jnp.float32),
            pltpu.VMEM_SHARED((NPD, D), jnp.float32),
            pltpu.SemaphoreType.DMA,
            pltpu.SemaphoreType.DMA,
            pltpu.SemaphoreType.DMA,
            pltpu.SemaphoreType.DMA,
        ],
    )
    def k(src3_h, dst3_h, w3_h, hp_hbm, s_out,
          sg_v, dg_v, wg_v, rows_a, rows_b, s_sh,
          semg_a, semg_b, sems_a, sems_b):
        c = lax.axis_index("c")
        s = lax.axis_index("s")
        wid = s * NC + c
        base = s * S_ROWS

        z16 = jnp.zeros((16,), jnp.float32)

        def zero_rows(i, carry):
            for t in range(D // 16):
                rows_a[i, pl.ds(t * 16, 16)] = z16
            return carry
        lax.fori_loop(0, K, zero_rows, 0)
        for kk in range(S_ROWS // K):
            pltpu.sync_copy(rows_a, s_sh.at[pl.ds(base + kk * K, K)])
        plsc.subcore_barrier()

        def g_start(u, buf, sem):
            pltpu.async_copy(hp_hbm.at[sg_v.at[u]], buf, sem)

        def g_wait(u, buf, sem):
            pltpu.make_async_copy(hp_hbm.at[sg_v.at[u]], buf, sem).wait()

        def s_start(u, buf, sem):
            pltpu.async_copy(buf, s_sh.at[dg_v.at[u]], sem, add=True)

        def s_wait(u, buf, sem):
            pltpu.make_async_copy(buf, s_sh.at[dg_v.at[u]], sem).wait()

        def scale(u, buf):
            def row(i, carry):
                ii = jnp.full((16,), i, jnp.int32)
                uu = jnp.full((16,), u, jnp.int32)
                cb = plsc.load_gather(wg_v, [uu, ii])
                for t in range(D // 16):
                    buf[i, pl.ds(t * 16, 16)] = buf[i, pl.ds(t * 16, 16)] * cb
                return carry
            lax.fori_loop(0, K, row, 0)

        for g in range(NG):
            pltpu.sync_copy(src3_h.at[wid, g], sg_v)
            pltpu.sync_copy(dst3_h.at[wid, g], dg_v)
            pltpu.sync_copy(w3_h.at[wid, g], wg_v)
            g_start(0, rows_a, semg_a)

            def pair(tt, carry):
                u0 = 2 * tt
                u1 = u0 + 1
                g_start(u1, rows_b, semg_b)
                g_wait(u0, rows_a, semg_a)
                scale(u0, rows_a)
                s_start(u0, rows_a, sems_a)
                g_wait(u1, rows_b, semg_b)
                scale(u1, rows_b)
                s_start(u1, rows_b, sems_b)
                s_wait(u0, rows_a, sems_a)
                g_start(u0 + 2, rows_a, semg_a)
                s_wait(u1, rows_b, sems_b)
                return carry
            lax.fori_loop(0, (GC - 1) // 2, pair, 0)

            ul = GC - 1
            g_wait(ul, rows_a, semg_a)
            scale(ul, rows_a)
            s_start(ul, rows_a, sems_a)
            s_wait(ul, rows_a, sems_a)
        plsc.subcore_barrier()

        pltpu.sync_copy(s_sh.at[pl.ds(base, S_ROWS)],
                        s_out.at[c, pl.ds(base, S_ROWS)])

    return k(src3, dst3, w3, hp)


def _mm_h(x, W_conv, deg0, deg1):
    def body(x_ref, w_ref, d0_ref, d1_ref, hp_ref, dis_ref):
        deg = d0_ref[...] + d1_ref[...] + 1.0
        dis = lax.rsqrt(deg)
        hp_ref[...] = jnp.dot(x_ref[...], w_ref[...],
                              preferred_element_type=jnp.float32) * dis
        dis_ref[...] = dis

    return pl.pallas_call(
        body,
        grid=(N // 400,),
        in_specs=[
            pl.BlockSpec((400, D), lambda i: (i, 0)),
            pl.BlockSpec((D, D), lambda i: (0, 0)),
            pl.BlockSpec((400, 1), lambda i: (i, 0)),
            pl.BlockSpec((400, 1), lambda i: (i, 0)),
        ],
        out_specs=[
            pl.BlockSpec((400, D), lambda i: (i, 0)),
            pl.BlockSpec((400, 1), lambda i: (i, 0)),
        ],
        out_shape=[
            jax.ShapeDtypeStruct((N, D), jnp.float32),
            jax.ShapeDtypeStruct((N, 1), jnp.float32),
        ],
    )(x, W_conv, deg0, deg1)


def _epilogue(s0, s1, hp, dis, b_conv, W_out, b_out):
    def body(s0_ref, s1_ref, hp_ref, dis_ref, bc_ref, wo_ref, bo_ref, o_ref):
        agg = dis_ref[...] * (s0_ref[...] + s1_ref[...] + hp_ref[...]) \
            + bc_ref[...]
        emb = jnp.where(agg >= 0, agg, 0.01 * agg)
        o_ref[...] = jnp.dot(emb, wo_ref[...],
                             preferred_element_type=jnp.float32) + bo_ref[...]

    return pl.pallas_call(
        body,
        grid=(N // 400,),
        in_specs=[
            pl.BlockSpec((400, D), lambda i: (i, 0)),
            pl.BlockSpec((400, D), lambda i: (i, 0)),
            pl.BlockSpec((400, D), lambda i: (i, 0)),
            pl.BlockSpec((400, 1), lambda i: (i, 0)),
            pl.BlockSpec((1, D), lambda i: (0, 0)),
            pl.BlockSpec((D, D_OUT), lambda i: (0, 0)),
            pl.BlockSpec((1, D_OUT), lambda i: (0, 0)),
        ],
        out_specs=pl.BlockSpec((400, D_OUT), lambda i: (i, 0)),
        out_shape=jax.ShapeDtypeStruct((N, D_OUT), jnp.float32),
    )(s0, s1, hp, dis, b_conv, W_out, b_out)


def kernel(x, edge_index, edge_weight, W_conv, b_conv, W_out, b_out):
    src = edge_index[0].astype(jnp.int32)
    dst = edge_index[1].astype(jnp.int32)
    w = edge_weight.astype(jnp.float32)

    src4 = src.reshape(NW, NG, GC, K)
    dst4 = dst.reshape(NW, NG, GC, K)
    w4 = w.reshape(NW, NG, GC, K)

    deg_p = _deg_kernel(dst4, w4)
    deg0 = deg_p[0, :N].reshape(N, 1)
    deg1 = deg_p[1, :N].reshape(N, 1)
    hp, dis = _mm_h(x, W_conv, deg0, deg1)
    s_part = _agg_kernel(src4, dst4, w4, hp)
    out = _epilogue(
        s_part[0, :N], s_part[1, :N], hp, dis,
        b_conv.reshape(1, D), W_out, b_out.reshape(1, D_OUT))
    return out
